# baseline (speedup 1.0000x reference)
import functools

import jax
import jax.numpy as jnp
from jax import lax
from jax.experimental import pallas as pl
from jax.experimental.pallas import tpu as pltpu, tpu_sc as plsc

N = 10000
E = 320000
D = 128
NW = 32
ET = E // NW
CK = 80
NCH = ET // CK
_NSEG = 5
CPS = NCH // _NSEG

NDEG = 10240
_DSLAB = NDEG // 16
NACC = 10240
_ASLAB = NACC // 16

_mesh = plsc.VectorSubcoreMesh(core_axis_name="c", subcore_axis_name="s")


@functools.partial(
    pl.kernel,
    mesh=_mesh,
    out_type=jax.ShapeDtypeStruct((2, NDEG), jnp.float32),
    scratch_types=[
        pltpu.VMEM((CPS, CK), jnp.int32),
        pltpu.VMEM((_DSLAB,), jnp.float32),
        pltpu.VMEM((CK,), jnp.float32),
        pltpu.VMEM_SHARED((NDEG,), jnp.float32),
    ],
)
def _deg_kernel(ei_hbm, deg_hbm, ridx_v, slab_v, ones_v, deg_sp):
    cid = lax.axis_index("c")
    sid = lax.axis_index("s")
    wid = sid * 2 + cid

    zeros16 = jnp.zeros((16,), jnp.float32)
    ones16 = jnp.ones((16,), jnp.float32)

    def zero_body(i, carry):
        slab_v[pl.ds(i * 16, 16)] = zeros16
        return carry

    lax.fori_loop(0, _DSLAB // 16, zero_body, 0)
    for j in range(CK // 16):
        ones_v[pl.ds(j * 16, 16)] = ones16

    pltpu.sync_copy(slab_v, deg_sp.at[pl.ds(sid * _DSLAB, _DSLAB)])
    plsc.subcore_barrier()

    def seg_body(seg, carry):
        pltpu.sync_copy(ei_hbm.at[0, wid, seg], ridx_v)

        def chunk_body(ch, carry2):
            pltpu.sync_copy(ones_v, deg_sp.at[ridx_v.at[ch]], add=True)
            return carry2

        lax.fori_loop(0, CPS, chunk_body, 0)
        return carry

    lax.fori_loop(0, _NSEG, seg_body, 0)
    plsc.subcore_barrier()

    pltpu.sync_copy(deg_sp.at[pl.ds(sid * _DSLAB, _DSLAB)], slab_v)
    pltpu.sync_copy(slab_v, deg_hbm.at[cid, pl.ds(sid * _DSLAB, _DSLAB)])


@functools.partial(
    pl.kernel,
    mesh=_mesh,
    out_type=jax.ShapeDtypeStruct((2, NACC, D), jnp.float32),
    scratch_types=[
        pltpu.VMEM((CPS, CK), jnp.int32),
        pltpu.VMEM((CPS, CK), jnp.int32),
        pltpu.VMEM((CK, D), jnp.float32),
        pltpu.VMEM((CK, D), jnp.float32),
        pltpu.VMEM((CK, D), jnp.float32),
        pltpu.VMEM((CK, D), jnp.float32),
        pltpu.VMEM_SHARED((NACC, D), jnp.float32),
        pltpu.SemaphoreType.DMA,
        pltpu.SemaphoreType.DMA,
        pltpu.SemaphoreType.DMA,
        pltpu.SemaphoreType.DMA,
    ],
)
def _scatter_kernel(g_hbm, ei_hbm, acc_hbm, ridx_v, cidx_v, buf0,
                    buf1, buf2, buf3, acc_sp, semg0, semg1, semg2, semg3):
    cid = lax.axis_index("c")
    sid = lax.axis_index("s")
    wid = sid * 2 + cid

    zeros16 = jnp.zeros((16,), jnp.float32)

    def zero_body(r, carry):
        for j in range(D // 16):
            buf0[r, pl.ds(j * 16, 16)] = zeros16
        return carry

    lax.fori_loop(0, CK, zero_body, 0)

    base = sid * _ASLAB
    for k in range(_ASLAB // CK):
        pltpu.sync_copy(buf0, acc_sp.at[pl.ds(base + k * CK, CK)])
    plsc.subcore_barrier()

    def gather(ch, buf, sem):
        pltpu.async_copy(g_hbm.at[ridx_v.at[ch]], buf, sem)

    def gwait(buf, sem):
        pltpu.make_async_copy(g_hbm.at[pl.ds(0, CK)], buf, sem).wait()

    def scat(ch, buf):
        pltpu.sync_copy(buf, acc_sp.at[cidx_v.at[ch]], add=True)

    def seg_body(seg, carry):
        pltpu.sync_copy(ei_hbm.at[0, wid, seg], ridx_v)
        pltpu.sync_copy(ei_hbm.at[1, wid, seg], cidx_v)
        gather(0, buf0, semg0)
        gather(1, buf1, semg1)
        gather(2, buf2, semg2)
        gather(3, buf3, semg3)

        def chunk_body(j, carry2):
            ch = 4 * j
            gwait(buf0, semg0)
            scat(ch, buf0)
            gather(ch + 4, buf0, semg0)
            gwait(buf1, semg1)
            scat(ch + 1, buf1)
            gather(ch + 5, buf1, semg1)
            gwait(buf2, semg2)
            scat(ch + 2, buf2)
            gather(ch + 6, buf2, semg2)
            gwait(buf3, semg3)
            scat(ch + 3, buf3)
            gather(ch + 7, buf3, semg3)
            return carry2

        lax.fori_loop(0, (CPS - 5) // 4, chunk_body, 0)
        gwait(buf0, semg0)
        scat(CPS - 5, buf0)
        gather(CPS - 1, buf0, semg0)
        gwait(buf1, semg1)
        scat(CPS - 4, buf1)
        gwait(buf2, semg2)
        scat(CPS - 3, buf2)
        gwait(buf3, semg3)
        scat(CPS - 2, buf3)
        gwait(buf0, semg0)
        scat(CPS - 1, buf0)
        return carry

    lax.fori_loop(0, _NSEG, seg_body, 0)
    plsc.subcore_barrier()

    for k in range(_ASLAB // CK):
        off = base + k * CK
        pltpu.sync_copy(acc_sp.at[pl.ds(off, CK)], buf0)
        pltpu.sync_copy(buf0, acc_hbm.at[cid, pl.ds(off, CK)])


_BLK = 2000
_GRID = N // _BLK


def _tc_a0_body(x_ref, wt_ref, b_ref, h_ref):
    h = jnp.dot(x_ref[...], wt_ref[...], preferred_element_type=jnp.float32)
    h_ref[...] = h + b_ref[...]


def _tc_a0(x, wt, b2):
    return pl.pallas_call(
        _tc_a0_body,
        grid=(_GRID,),
        in_specs=[
            pl.BlockSpec((_BLK, D), lambda i: (i, 0)),
            pl.BlockSpec((D, D), lambda i: (0, 0)),
            pl.BlockSpec((1, D), lambda i: (0, 0)),
        ],
        out_specs=pl.BlockSpec((_BLK, D), lambda i: (i, 0)),
        out_shape=jax.ShapeDtypeStruct((N, D), jnp.float32),
    )(x, wt, b2)


def _tc_a1_body(h_ref, d0_ref, d1_ref, g_ref, dis_ref):
    deg = d0_ref[0] + d1_ref[0] + 1.0
    dis = lax.rsqrt(deg)
    g_ref[...] = dis * h_ref[...]
    dis_ref[...] = dis


def _tc_a1(h, degp3):
    return pl.pallas_call(
        _tc_a1_body,
        grid=(_GRID,),
        in_specs=[
            pl.BlockSpec((_BLK, D), lambda i: (i, 0)),
            pl.BlockSpec((1, _BLK, 1), lambda i: (0, i, 0)),
            pl.BlockSpec((1, _BLK, 1), lambda i: (1, i, 0)),
        ],
        out_specs=[
            pl.BlockSpec((_BLK, D), lambda i: (i, 0)),
            pl.BlockSpec((_BLK, 1), lambda i: (i, 0)),
        ],
        out_shape=[
            jax.ShapeDtypeStruct((N, D), jnp.float32),
            jax.ShapeDtypeStruct((N, 1), jnp.float32),
        ],
    )(h, degp3, degp3)


def _tc_b_body(a0_ref, a1_ref, g_ref, dis_ref, out_ref):
    out_ref[...] = dis_ref[...] * (a0_ref[0] + a1_ref[0] + g_ref[...])


def _tc_b(acc, g, dis):
    return pl.pallas_call(
        _tc_b_body,
        grid=(_GRID,),
        in_specs=[
            pl.BlockSpec((1, _BLK, D), lambda i: (0, i, 0)),
            pl.BlockSpec((1, _BLK, D), lambda i: (1, i, 0)),
            pl.BlockSpec((_BLK, D), lambda i: (i, 0)),
            pl.BlockSpec((_BLK, 1), lambda i: (i, 0)),
        ],
        out_specs=pl.BlockSpec((_BLK, D), lambda i: (i, 0)),
        out_shape=jax.ShapeDtypeStruct((N, D), jnp.float32),
    )(acc, acc, g, dis)


def kernel(x, edge_index, W, b):
    ei5 = edge_index.reshape(2, NW, _NSEG, CPS, CK)

    degp = _deg_kernel(ei5)
    degp3 = degp.reshape(2, NDEG, 1)

    h = _tc_a0(x, W.T, b.reshape(1, D))
    g, dis = _tc_a1(h, degp3)
    acc = _scatter_kernel(g, ei5)
    return _tc_b(acc, g, dis)


# speedup vs baseline: 1.0136x; 1.0136x over previous
import functools

import jax
import jax.numpy as jnp
from jax import lax
from jax.experimental import pallas as pl
from jax.experimental.pallas import tpu as pltpu, tpu_sc as plsc

N = 10000
E = 320000
D = 128
NW = 32
ET = E // NW
CK = 80
NCH = ET // CK
_NSEG = 5
CPS = NCH // _NSEG

NDEG = 10240
_DSLAB = NDEG // 16
NACC = 10240
_ASLAB = NACC // 16

_mesh = plsc.VectorSubcoreMesh(core_axis_name="c", subcore_axis_name="s")


@functools.partial(
    pl.kernel,
    mesh=_mesh,
    out_type=jax.ShapeDtypeStruct((2, NDEG), jnp.float32),
    scratch_types=[
        pltpu.VMEM((CPS, CK), jnp.int32),
        pltpu.VMEM((_DSLAB,), jnp.float32),
        pltpu.VMEM((CK,), jnp.float32),
        pltpu.VMEM_SHARED((NDEG,), jnp.float32),
    ],
)
def _deg_kernel(ei_hbm, deg_hbm, ridx_v, slab_v, ones_v, deg_sp):
    cid = lax.axis_index("c")
    sid = lax.axis_index("s")
    wid = sid * 2 + cid

    zeros16 = jnp.zeros((16,), jnp.float32)
    ones16 = jnp.ones((16,), jnp.float32)

    def zero_body(i, carry):
        slab_v[pl.ds(i * 16, 16)] = zeros16
        return carry

    lax.fori_loop(0, _DSLAB // 16, zero_body, 0)
    for j in range(CK // 16):
        ones_v[pl.ds(j * 16, 16)] = ones16

    pltpu.sync_copy(slab_v, deg_sp.at[pl.ds(sid * _DSLAB, _DSLAB)])
    plsc.subcore_barrier()

    def seg_body(seg, carry):
        pltpu.sync_copy(ei_hbm.at[0, wid, seg], ridx_v)

        def chunk_body(ch, carry2):
            pltpu.sync_copy(ones_v, deg_sp.at[ridx_v.at[ch]], add=True)
            return carry2

        lax.fori_loop(0, CPS, chunk_body, 0)
        return carry

    lax.fori_loop(0, _NSEG, seg_body, 0)
    plsc.subcore_barrier()

    pltpu.sync_copy(deg_sp.at[pl.ds(sid * _DSLAB, _DSLAB)], slab_v)
    pltpu.sync_copy(slab_v, deg_hbm.at[cid, pl.ds(sid * _DSLAB, _DSLAB)])


@functools.partial(
    pl.kernel,
    mesh=_mesh,
    out_type=jax.ShapeDtypeStruct((2, NACC, D), jnp.float32),
    scratch_types=[
        pltpu.VMEM((CPS, CK), jnp.int32),
        pltpu.VMEM((CPS, CK), jnp.int32),
        pltpu.VMEM((CK, D), jnp.float32),
        pltpu.VMEM((CK, D), jnp.float32),
        pltpu.VMEM((CK, D), jnp.float32),
        pltpu.VMEM((CK, D), jnp.float32),
        pltpu.VMEM_SHARED((NACC, D), jnp.float32),
        pltpu.SemaphoreType.DMA,
        pltpu.SemaphoreType.DMA,
        pltpu.SemaphoreType.DMA,
        pltpu.SemaphoreType.DMA,
    ],
)
def _scatter_kernel(g_hbm, ei_hbm, acc_hbm, ridx_v, cidx_v, buf0,
                    buf1, buf2, buf3, acc_sp, semg0, semg1, semg2, semg3):
    cid = lax.axis_index("c")
    sid = lax.axis_index("s")
    wid = sid * 2 + cid

    zeros16 = jnp.zeros((16,), jnp.float32)

    def zero_body(r, carry):
        for j in range(D // 16):
            buf0[r, pl.ds(j * 16, 16)] = zeros16
        return carry

    lax.fori_loop(0, CK, zero_body, 0)

    base = sid * _ASLAB
    for k in range(_ASLAB // CK):
        pltpu.sync_copy(buf0, acc_sp.at[pl.ds(base + k * CK, CK)])
    plsc.subcore_barrier()

    def gather(ch, buf, sem):
        pltpu.async_copy(g_hbm.at[ridx_v.at[ch]], buf, sem)

    def gwait(buf, sem):
        pltpu.make_async_copy(g_hbm.at[pl.ds(0, CK)], buf, sem).wait()

    def scat(ch, buf):
        pltpu.sync_copy(buf, acc_sp.at[cidx_v.at[ch]], add=True)

    def seg_body(seg, carry):
        pltpu.sync_copy(ei_hbm.at[0, wid, seg], ridx_v)
        pltpu.sync_copy(ei_hbm.at[1, wid, seg], cidx_v)
        gather(0, buf0, semg0)
        gather(1, buf1, semg1)
        gather(2, buf2, semg2)
        gather(3, buf3, semg3)

        def chunk_body(j, carry2):
            ch = 4 * j
            gwait(buf0, semg0)
            scat(ch, buf0)
            gather(ch + 4, buf0, semg0)
            gwait(buf1, semg1)
            scat(ch + 1, buf1)
            gather(ch + 5, buf1, semg1)
            gwait(buf2, semg2)
            scat(ch + 2, buf2)
            gather(ch + 6, buf2, semg2)
            gwait(buf3, semg3)
            scat(ch + 3, buf3)
            gather(ch + 7, buf3, semg3)
            return carry2

        lax.fori_loop(0, (CPS - 5) // 4, chunk_body, 0)
        gwait(buf0, semg0)
        scat(CPS - 5, buf0)
        gather(CPS - 1, buf0, semg0)
        gwait(buf1, semg1)
        scat(CPS - 4, buf1)
        gwait(buf2, semg2)
        scat(CPS - 3, buf2)
        gwait(buf3, semg3)
        scat(CPS - 2, buf3)
        gwait(buf0, semg0)
        scat(CPS - 1, buf0)
        return carry

    lax.fori_loop(0, _NSEG, seg_body, 0)
    plsc.subcore_barrier()

    for k in range(_ASLAB // CK):
        off = base + k * CK
        pltpu.sync_copy(acc_sp.at[pl.ds(off, CK)], buf0)
        pltpu.sync_copy(buf0, acc_hbm.at[cid, pl.ds(off, CK)])


_BLK = 5000
_GRID = N // _BLK


def _tc_a_body(x_ref, wt_ref, b_ref, d0_ref, d1_ref, g_ref, dis_ref):
    h = jnp.dot(x_ref[...], wt_ref[...], preferred_element_type=jnp.float32)
    h = h + b_ref[...]
    deg = d0_ref[0] + d1_ref[0] + 1.0
    dis = lax.rsqrt(deg)
    g_ref[...] = dis * h
    dis_ref[...] = dis


def _tc_a(x, wt, b2, degp3):
    return pl.pallas_call(
        _tc_a_body,
        grid=(_GRID,),
        in_specs=[
            pl.BlockSpec((_BLK, D), lambda i: (i, 0)),
            pl.BlockSpec((D, D), lambda i: (0, 0)),
            pl.BlockSpec((1, D), lambda i: (0, 0)),
            pl.BlockSpec((1, _BLK, 1), lambda i: (0, i, 0)),
            pl.BlockSpec((1, _BLK, 1), lambda i: (1, i, 0)),
        ],
        out_specs=[
            pl.BlockSpec((_BLK, D), lambda i: (i, 0)),
            pl.BlockSpec((_BLK, 1), lambda i: (i, 0)),
        ],
        out_shape=[
            jax.ShapeDtypeStruct((N, D), jnp.float32),
            jax.ShapeDtypeStruct((N, 1), jnp.float32),
        ],
    )(x, wt, b2, degp3, degp3)


def _tc_b_body(a0_ref, a1_ref, g_ref, dis_ref, out_ref):
    out_ref[...] = dis_ref[...] * (a0_ref[0] + a1_ref[0] + g_ref[...])


def _tc_b(acc, g, dis):
    return pl.pallas_call(
        _tc_b_body,
        grid=(_GRID,),
        in_specs=[
            pl.BlockSpec((1, _BLK, D), lambda i: (0, i, 0)),
            pl.BlockSpec((1, _BLK, D), lambda i: (1, i, 0)),
            pl.BlockSpec((_BLK, D), lambda i: (i, 0)),
            pl.BlockSpec((_BLK, 1), lambda i: (i, 0)),
        ],
        out_specs=pl.BlockSpec((_BLK, D), lambda i: (i, 0)),
        out_shape=jax.ShapeDtypeStruct((N, D), jnp.float32),
    )(acc, acc, g, dis)


def kernel(x, edge_index, W, b):
    ei5 = edge_index.reshape(2, NW, _NSEG, CPS, CK)

    degp = _deg_kernel(ei5)
    degp3 = degp.reshape(2, NDEG, 1)

    g, dis = _tc_a(x, W.T, b.reshape(1, D), degp3)
    acc = _scatter_kernel(g, ei5)
    return _tc_b(acc, g, dis)


# speedup vs baseline: 1.0591x; 1.0449x over previous
import functools

import jax
import jax.numpy as jnp
from jax import lax
from jax.experimental import pallas as pl
from jax.experimental.pallas import tpu as pltpu, tpu_sc as plsc

N = 10000
E = 320000
D = 128
NW = 32
ET = E // NW
CK = 80
NCH = ET // CK
_NSEG = 5
CPS = NCH // _NSEG

NDEG = 10240
_DSLAB = NDEG // 16
NACC = 10240
_ASLAB = NACC // 16

_mesh = plsc.VectorSubcoreMesh(core_axis_name="c", subcore_axis_name="s")


@functools.partial(
    pl.kernel,
    mesh=_mesh,
    out_type=jax.ShapeDtypeStruct((2, NDEG), jnp.float32),
    scratch_types=[
        pltpu.VMEM((CPS, CK), jnp.int32),
        pltpu.VMEM((_DSLAB,), jnp.float32),
        pltpu.VMEM((CK,), jnp.float32),
        pltpu.VMEM_SHARED((NDEG,), jnp.float32),
        pltpu.SemaphoreType.DMA,
    ],
)
def _deg_kernel(ei_hbm, deg_hbm, ridx_v, slab_v, ones_v, deg_sp, sem):
    cid = lax.axis_index("c")
    sid = lax.axis_index("s")
    wid = sid * 2 + cid

    zeros16 = jnp.zeros((16,), jnp.float32)
    ones16 = jnp.ones((16,), jnp.float32)

    def zero_body(i, carry):
        slab_v[pl.ds(i * 16, 16)] = zeros16
        return carry

    lax.fori_loop(0, _DSLAB // 16, zero_body, 0)
    for j in range(CK // 16):
        ones_v[pl.ds(j * 16, 16)] = ones16

    pltpu.sync_copy(slab_v, deg_sp.at[pl.ds(sid * _DSLAB, _DSLAB)])
    plsc.subcore_barrier()

    def seg_body(seg, carry):
        pltpu.sync_copy(ei_hbm.at[0, wid, seg], ridx_v)

        def chunk_body(ch, carry2):
            pltpu.async_copy(ones_v, deg_sp.at[ridx_v.at[ch]], sem, add=True)
            return carry2

        lax.fori_loop(0, CPS, chunk_body, 0)

        def drain_body(ch, carry2):
            pltpu.make_async_copy(deg_hbm.at[cid, pl.ds(0, CK)], ones_v,
                                  sem).wait()
            return carry2

        lax.fori_loop(0, CPS, drain_body, 0)
        return carry

    lax.fori_loop(0, _NSEG, seg_body, 0)
    plsc.subcore_barrier()

    pltpu.sync_copy(deg_sp.at[pl.ds(sid * _DSLAB, _DSLAB)], slab_v)
    pltpu.sync_copy(slab_v, deg_hbm.at[cid, pl.ds(sid * _DSLAB, _DSLAB)])


@functools.partial(
    pl.kernel,
    mesh=_mesh,
    out_type=jax.ShapeDtypeStruct((2, NACC, D), jnp.float32),
    scratch_types=[
        pltpu.VMEM((CPS, CK), jnp.int32),
        pltpu.VMEM((CPS, CK), jnp.int32),
        pltpu.VMEM((CK, D), jnp.float32),
        pltpu.VMEM((CK, D), jnp.float32),
        pltpu.VMEM((CK, D), jnp.float32),
        pltpu.VMEM((CK, D), jnp.float32),
        pltpu.VMEM_SHARED((NACC, D), jnp.float32),
        pltpu.SemaphoreType.DMA,
        pltpu.SemaphoreType.DMA,
        pltpu.SemaphoreType.DMA,
        pltpu.SemaphoreType.DMA,
    ],
)
def _scatter_kernel(g_hbm, ei_hbm, acc_hbm, ridx_v, cidx_v, buf0,
                    buf1, buf2, buf3, acc_sp, semg0, semg1, semg2, semg3):
    cid = lax.axis_index("c")
    sid = lax.axis_index("s")
    wid = sid * 2 + cid

    zeros16 = jnp.zeros((16,), jnp.float32)

    def zero_body(r, carry):
        for j in range(D // 16):
            buf0[r, pl.ds(j * 16, 16)] = zeros16
        return carry

    lax.fori_loop(0, CK, zero_body, 0)

    base = sid * _ASLAB
    for k in range(_ASLAB // CK):
        pltpu.sync_copy(buf0, acc_sp.at[pl.ds(base + k * CK, CK)])
    plsc.subcore_barrier()

    def gather(ch, buf, sem):
        pltpu.async_copy(g_hbm.at[ridx_v.at[ch]], buf, sem)

    def gwait(buf, sem):
        pltpu.make_async_copy(g_hbm.at[pl.ds(0, CK)], buf, sem).wait()

    def scat(ch, buf):
        pltpu.sync_copy(buf, acc_sp.at[cidx_v.at[ch]], add=True)

    def seg_body(seg, carry):
        pltpu.sync_copy(ei_hbm.at[0, wid, seg], ridx_v)
        pltpu.sync_copy(ei_hbm.at[1, wid, seg], cidx_v)
        gather(0, buf0, semg0)
        gather(1, buf1, semg1)
        gather(2, buf2, semg2)
        gather(3, buf3, semg3)

        def chunk_body(j, carry2):
            ch = 4 * j
            gwait(buf0, semg0)
            scat(ch, buf0)
            gather(ch + 4, buf0, semg0)
            gwait(buf1, semg1)
            scat(ch + 1, buf1)
            gather(ch + 5, buf1, semg1)
            gwait(buf2, semg2)
            scat(ch + 2, buf2)
            gather(ch + 6, buf2, semg2)
            gwait(buf3, semg3)
            scat(ch + 3, buf3)
            gather(ch + 7, buf3, semg3)
            return carry2

        lax.fori_loop(0, (CPS - 5) // 4, chunk_body, 0)
        gwait(buf0, semg0)
        scat(CPS - 5, buf0)
        gather(CPS - 1, buf0, semg0)
        gwait(buf1, semg1)
        scat(CPS - 4, buf1)
        gwait(buf2, semg2)
        scat(CPS - 3, buf2)
        gwait(buf3, semg3)
        scat(CPS - 2, buf3)
        gwait(buf0, semg0)
        scat(CPS - 1, buf0)
        return carry

    lax.fori_loop(0, _NSEG, seg_body, 0)
    plsc.subcore_barrier()

    for k in range(_ASLAB // CK):
        off = base + k * CK
        pltpu.sync_copy(acc_sp.at[pl.ds(off, CK)], buf0)
        pltpu.sync_copy(buf0, acc_hbm.at[cid, pl.ds(off, CK)])


_BLK = 5000
_GRID = N // _BLK


def _tc_a_body(x_ref, wt_ref, b_ref, d0_ref, d1_ref, g_ref, dis_ref):
    h = jnp.dot(x_ref[...], wt_ref[...], preferred_element_type=jnp.float32)
    h = h + b_ref[...]
    deg = d0_ref[0] + d1_ref[0] + 1.0
    dis = lax.rsqrt(deg)
    g_ref[...] = dis * h
    dis_ref[...] = dis


def _tc_a(x, wt, b2, degp3):
    return pl.pallas_call(
        _tc_a_body,
        grid=(_GRID,),
        in_specs=[
            pl.BlockSpec((_BLK, D), lambda i: (i, 0)),
            pl.BlockSpec((D, D), lambda i: (0, 0)),
            pl.BlockSpec((1, D), lambda i: (0, 0)),
            pl.BlockSpec((1, _BLK, 1), lambda i: (0, i, 0)),
            pl.BlockSpec((1, _BLK, 1), lambda i: (1, i, 0)),
        ],
        out_specs=[
            pl.BlockSpec((_BLK, D), lambda i: (i, 0)),
            pl.BlockSpec((_BLK, 1), lambda i: (i, 0)),
        ],
        out_shape=[
            jax.ShapeDtypeStruct((N, D), jnp.float32),
            jax.ShapeDtypeStruct((N, 1), jnp.float32),
        ],
    )(x, wt, b2, degp3, degp3)


def _tc_b_body(a0_ref, a1_ref, g_ref, dis_ref, out_ref):
    out_ref[...] = dis_ref[...] * (a0_ref[0] + a1_ref[0] + g_ref[...])


def _tc_b(acc, g, dis):
    return pl.pallas_call(
        _tc_b_body,
        grid=(_GRID,),
        in_specs=[
            pl.BlockSpec((1, _BLK, D), lambda i: (0, i, 0)),
            pl.BlockSpec((1, _BLK, D), lambda i: (1, i, 0)),
            pl.BlockSpec((_BLK, D), lambda i: (i, 0)),
            pl.BlockSpec((_BLK, 1), lambda i: (i, 0)),
        ],
        out_specs=pl.BlockSpec((_BLK, D), lambda i: (i, 0)),
        out_shape=jax.ShapeDtypeStruct((N, D), jnp.float32),
    )(acc, acc, g, dis)


def kernel(x, edge_index, W, b):
    ei5 = edge_index.reshape(2, NW, _NSEG, CPS, CK)

    degp = _deg_kernel(ei5)
    degp3 = degp.reshape(2, NDEG, 1)

    g, dis = _tc_a(x, W.T, b.reshape(1, D), degp3)
    acc = _scatter_kernel(g, ei5)
    return _tc_b(acc, g, dis)
